# Initial kernel scaffold; baseline (speedup 1.0000x reference)
import jax
import jax.numpy as jnp
from jax.experimental import pallas as pl


def kernel(x, edge_index, batch, W1, b1, W2, b2, W3, b3, Wc, bc):
    raise NotImplementedError("write your pallas kernel here")


# speedup vs baseline: 20.4861x; 20.4861x over previous
import functools

import jax
import jax.numpy as jnp
from jax import lax
from jax.experimental import pallas as pl
from jax.experimental.pallas import tpu as pltpu
from jax.experimental.pallas import tpu_sc as plsc

N = 10000
NPAD = 10240
E = 160000
EPAD = 163840
G = 64
NW = 32
ROWS_PER_W = EPAD // (NW * 128)
RPT = NPAD // 16
BN = 1024
NBLK = NPAD // BN

_mesh = plsc.VectorSubcoreMesh(core_axis_name="c", subcore_axis_name="s")


def _deg_body(dst_hbm, zeros_hbm, ones_hbm, out_hbm, idx_v, ones_v, acc, sem):
    cid = lax.axis_index("c")
    sid = lax.axis_index("s")
    wid = sid * 2 + cid
    pltpu.sync_copy(zeros_hbm.at[pl.ds(sid * RPT, RPT)],
                    acc.at[pl.ds(sid * RPT, RPT)])
    pltpu.sync_copy(ones_hbm, ones_v)
    pltpu.sync_copy(dst_hbm.at[pl.ds(wid * ROWS_PER_W, ROWS_PER_W)], idx_v)
    plsc.subcore_barrier()

    def body(j, carry):
        pltpu.sync_copy(ones_v, acc.at[idx_v.at[j]], add=True)
        return carry

    lax.fori_loop(0, ROWS_PER_W, body, 0)
    plsc.subcore_barrier()
    pltpu.sync_copy(acc.at[pl.ds(sid * RPT, RPT)],
                    out_hbm.at[cid, pl.ds(sid * RPT, RPT)])


def _sc_degree(dst2d, zeros, ones):
    return pl.kernel(
        _deg_body,
        out_type=jax.ShapeDtypeStruct((2, NPAD, 16), jnp.float32),
        mesh=_mesh,
        scratch_types=[
            pltpu.VMEM((ROWS_PER_W, 128), jnp.int32),
            pltpu.VMEM((128, 16), jnp.float32),
            pltpu.VMEM_SHARED((NPAD, 16), jnp.float32),
            pltpu.SemaphoreType.DMA,
        ],
        compiler_params=pltpu.CompilerParams(use_tc_tiling_on_sc=False),
    )(dst2d, zeros, ones)


def _prop_body(hs_hbm, src_hbm, dst_hbm, out_hbm, isv, idv, r0, r1, acc,
               s0, s1):
    cid = lax.axis_index("c")
    sid = lax.axis_index("s")
    wid = sid * 2 + cid
    pltpu.sync_copy(hs_hbm.at[pl.ds(sid * RPT, RPT)],
                    acc.at[pl.ds(sid * RPT, RPT)])
    pltpu.sync_copy(src_hbm.at[pl.ds(wid * ROWS_PER_W, ROWS_PER_W)], isv)
    pltpu.sync_copy(dst_hbm.at[pl.ds(wid * ROWS_PER_W, ROWS_PER_W)], idv)
    plsc.subcore_barrier()

    pltpu.async_copy(hs_hbm.at[isv.at[0]], r0, s0)

    def body(i, carry):
        j0 = 2 * i
        j1 = 2 * i + 1
        cp1 = pltpu.async_copy(hs_hbm.at[isv.at[j1]], r1, s1)
        pltpu.make_async_copy(hs_hbm.at[isv.at[j0]], r0, s0).wait()
        pltpu.sync_copy(r0, acc.at[idv.at[j0]], add=True)

        @pl.when(j0 + 2 < ROWS_PER_W)
        def _():
            pltpu.async_copy(hs_hbm.at[isv.at[j0 + 2]], r0, s0)

        cp1.wait()
        pltpu.sync_copy(r1, acc.at[idv.at[j1]], add=True)
        return carry

    lax.fori_loop(0, ROWS_PER_W // 2, body, 0)
    plsc.subcore_barrier()
    pltpu.sync_copy(acc.at[pl.ds(sid * RPT, RPT)],
                    out_hbm.at[cid, pl.ds(sid * RPT, RPT)])


def _sc_prop(hs, src2d, dst2d, D):
    return pl.kernel(
        _prop_body,
        out_type=jax.ShapeDtypeStruct((2, NPAD, D), jnp.float32),
        mesh=_mesh,
        scratch_types=[
            pltpu.VMEM((ROWS_PER_W, 128), jnp.int32),
            pltpu.VMEM((ROWS_PER_W, 128), jnp.int32),
            pltpu.VMEM((128, D), jnp.float32),
            pltpu.VMEM((128, D), jnp.float32),
            pltpu.VMEM_SHARED((NPAD, D), jnp.float32),
            pltpu.SemaphoreType.DMA,
            pltpu.SemaphoreType.DMA,
        ],
        compiler_params=pltpu.CompilerParams(use_tc_tiling_on_sc=False),
    )(hs, src2d, dst2d)


def _tc1_body(x_ref, w_ref, d0_ref, d1_ref, hs_ref, dinv_ref):
    deg = d0_ref[:, 0:1] + d1_ref[:, 0:1] + 1.0
    dinv = lax.rsqrt(deg)
    h = jnp.dot(x_ref[...], w_ref[...], preferred_element_type=jnp.float32)
    hs_ref[...] = h * dinv
    dinv_ref[...] = dinv


def _tc1(xp, W1, deg0, deg1):
    return pl.pallas_call(
        _tc1_body,
        grid=(NBLK,),
        in_specs=[
            pl.BlockSpec((BN, 256), lambda i: (i, 0)),
            pl.BlockSpec((256, 128), lambda i: (0, 0)),
            pl.BlockSpec((BN, 16), lambda i: (i, 0)),
            pl.BlockSpec((BN, 16), lambda i: (i, 0)),
        ],
        out_specs=[
            pl.BlockSpec((BN, 128), lambda i: (i, 0)),
            pl.BlockSpec((BN, 1), lambda i: (i, 0)),
        ],
        out_shape=[
            jax.ShapeDtypeStruct((NPAD, 128), jnp.float32),
            jax.ShapeDtypeStruct((NPAD, 1), jnp.float32),
        ],
    )(xp, W1, deg0, deg1)


def _tc_mid_body(a0_ref, a1_ref, hs_ref, dinv_ref, b_ref, w_ref, out_ref):
    dinv = dinv_ref[...]
    p = jnp.tanh(dinv * (a0_ref[...] + a1_ref[...] - hs_ref[...]) + b_ref[...])
    out_ref[...] = jnp.dot(p, w_ref[...],
                           preferred_element_type=jnp.float32) * dinv


def _tc_mid(a0, a1, hs, dinv, b, W, Din, Dout):
    return pl.pallas_call(
        _tc_mid_body,
        grid=(NBLK,),
        in_specs=[
            pl.BlockSpec((BN, Din), lambda i: (i, 0)),
            pl.BlockSpec((BN, Din), lambda i: (i, 0)),
            pl.BlockSpec((BN, Din), lambda i: (i, 0)),
            pl.BlockSpec((BN, 1), lambda i: (i, 0)),
            pl.BlockSpec((1, Din), lambda i: (0, 0)),
            pl.BlockSpec((Din, Dout), lambda i: (0, 0)),
        ],
        out_specs=pl.BlockSpec((BN, Dout), lambda i: (i, 0)),
        out_shape=jax.ShapeDtypeStruct((NPAD, Dout), jnp.float32),
    )(a0, a1, hs, dinv, b, W)


def _tc_final_body(a0_ref, a1_ref, hs_ref, dinv_ref, b_ref, batch_ref,
                   mask_ref, wc_ref, bc_ref, out_ref, h_ref, sacc):
    i = pl.program_id(0)

    @pl.when(i == 0)
    def _():
        sacc[...] = jnp.zeros_like(sacc)

    dinv = dinv_ref[...]
    p = jnp.tanh(dinv * (a0_ref[...] + a1_ref[...] - hs_ref[...]) + b_ref[...])
    paug = jnp.concatenate([p, jnp.ones((BN, 1), jnp.float32)], axis=1)
    iota = lax.broadcasted_iota(jnp.int32, (BN, G), 1)
    onehot = (batch_ref[...] == iota).astype(jnp.float32)
    sacc[...] += lax.dot_general(onehot, paug, (((0,), (0,)), ((), ())),
                                 preferred_element_type=jnp.float32)

    @pl.when(i == NBLK - 1)
    def _():
        s = sacc[...]
        hp = s[:, 0:16] / jnp.maximum(s[:, 16:17], 1.0)
        hd = mask_ref[...] * (2.0 * hp)
        h_ref[...] = hd
        out_ref[...] = jnp.dot(hd, wc_ref[...],
                               preferred_element_type=jnp.float32) + bc_ref[...]


def _tc_final(a0, a1, hs, dinv, b, batch2d, mask, Wc, bc):
    return pl.pallas_call(
        _tc_final_body,
        grid=(NBLK,),
        in_specs=[
            pl.BlockSpec((BN, 16), lambda i: (i, 0)),
            pl.BlockSpec((BN, 16), lambda i: (i, 0)),
            pl.BlockSpec((BN, 16), lambda i: (i, 0)),
            pl.BlockSpec((BN, 1), lambda i: (i, 0)),
            pl.BlockSpec((1, 16), lambda i: (0, 0)),
            pl.BlockSpec((BN, 1), lambda i: (i, 0)),
            pl.BlockSpec((G, 16), lambda i: (0, 0)),
            pl.BlockSpec((16, 2), lambda i: (0, 0)),
            pl.BlockSpec((1, 2), lambda i: (0, 0)),
        ],
        out_specs=[
            pl.BlockSpec((G, 2), lambda i: (0, 0)),
            pl.BlockSpec((G, 16), lambda i: (0, 0)),
        ],
        out_shape=[
            jax.ShapeDtypeStruct((G, 2), jnp.float32),
            jax.ShapeDtypeStruct((G, 16), jnp.float32),
        ],
        scratch_shapes=[pltpu.VMEM((G, 17), jnp.float32)],
    )(a0, a1, hs, dinv, b, batch2d, mask, Wc, bc)


def kernel(x, edge_index, batch, W1, b1, W2, b2, W3, b3, Wc, bc):
    xp = jnp.zeros((NPAD, 256), jnp.float32).at[:N].set(x)
    src = edge_index[0].astype(jnp.int32)
    dst = edge_index[1].astype(jnp.int32)
    npe = EPAD - E
    pad_idx = N + (jnp.arange(npe, dtype=jnp.int32) % (NPAD - N))
    src2d = jnp.concatenate([src, pad_idx]).reshape(EPAD // 128, 128)
    dst2d = jnp.concatenate([dst, pad_idx]).reshape(EPAD // 128, 128)
    batch2d = jnp.concatenate(
        [batch.astype(jnp.int32),
         jnp.full((NPAD - N,), G, jnp.int32)]).reshape(NPAD, 1)
    zeros16 = jnp.zeros((NPAD, 16), jnp.float32)
    ones16 = jnp.ones((128, 16), jnp.float32)
    mask = jax.random.bernoulli(jax.random.key(42), 0.5,
                                (G, 16)).astype(jnp.float32)
    b1r = b1.reshape(1, 128)
    b2r = b2.reshape(1, 64)
    b3r = b3.reshape(1, 16)
    bcr = bc.reshape(1, 2)

    degp = _sc_degree(dst2d, zeros16, ones16)
    hs1, dinv = _tc1(xp, W1, degp[0], degp[1])
    acc1 = _sc_prop(hs1, src2d, dst2d, 128)
    hs2 = _tc_mid(acc1[0], acc1[1], hs1, dinv, b1r, W2, 128, 64)
    acc2 = _sc_prop(hs2, src2d, dst2d, 64)
    hs3 = _tc_mid(acc2[0], acc2[1], hs2, dinv, b2r, W3, 64, 16)
    acc3 = _sc_prop(hs3, src2d, dst2d, 16)
    out2d, h = _tc_final(acc3[0], acc3[1], hs3, dinv, b3r, batch2d, mask,
                         Wc, bcr)
    return (out2d.reshape(-1), h)


# speedup vs baseline: 23.3266x; 1.1387x over previous
import functools

import jax
import jax.numpy as jnp
from jax import lax
from jax.experimental import pallas as pl
from jax.experimental.pallas import tpu as pltpu
from jax.experimental.pallas import tpu_sc as plsc

N = 10000
NPAD = 10240
E = 160000
EPAD = 163840
G = 64
NW = 32
ROWS_PER_W = EPAD // (NW * 128)
RPT = NPAD // 16

_mesh = plsc.VectorSubcoreMesh(core_axis_name="c", subcore_axis_name="s")


def _deg_body(dst_hbm, zeros_hbm, ones_hbm, out_hbm, idx_v, ones_v, acc, sem):
    cid = lax.axis_index("c")
    sid = lax.axis_index("s")
    wid = sid * 2 + cid
    pltpu.sync_copy(zeros_hbm.at[pl.ds(sid * RPT, RPT)],
                    acc.at[pl.ds(sid * RPT, RPT)])
    pltpu.sync_copy(ones_hbm, ones_v)
    pltpu.sync_copy(dst_hbm.at[pl.ds(wid * ROWS_PER_W, ROWS_PER_W)], idx_v)
    plsc.subcore_barrier()

    def body(j, carry):
        pltpu.sync_copy(ones_v, acc.at[idx_v.at[j]], add=True)
        return carry

    lax.fori_loop(0, ROWS_PER_W, body, 0)
    plsc.subcore_barrier()
    pltpu.sync_copy(acc.at[pl.ds(sid * RPT, RPT)],
                    out_hbm.at[cid, pl.ds(sid * RPT, RPT)])


def _sc_degree(dst2d, zeros, ones):
    return pl.kernel(
        _deg_body,
        out_type=jax.ShapeDtypeStruct((2, NPAD, 16), jnp.float32),
        mesh=_mesh,
        scratch_types=[
            pltpu.VMEM((ROWS_PER_W, 128), jnp.int32),
            pltpu.VMEM((128, 16), jnp.float32),
            pltpu.VMEM_SHARED((NPAD, 16), jnp.float32),
            pltpu.SemaphoreType.DMA,
        ],
        compiler_params=pltpu.CompilerParams(use_tc_tiling_on_sc=False),
    )(dst2d, zeros, ones)


def _prop_body(hs_hbm, src_hbm, dst_hbm, out_hbm, isv, idv, r0, r1, acc,
               s0, s1):
    cid = lax.axis_index("c")
    sid = lax.axis_index("s")
    wid = sid * 2 + cid
    pltpu.sync_copy(hs_hbm.at[pl.ds(sid * RPT, RPT)],
                    acc.at[pl.ds(sid * RPT, RPT)])
    pltpu.sync_copy(src_hbm.at[pl.ds(wid * ROWS_PER_W, ROWS_PER_W)], isv)
    pltpu.sync_copy(dst_hbm.at[pl.ds(wid * ROWS_PER_W, ROWS_PER_W)], idv)
    plsc.subcore_barrier()

    pltpu.async_copy(hs_hbm.at[isv.at[0]], r0, s0)

    def body(i, carry):
        j0 = 2 * i
        j1 = 2 * i + 1
        cp1 = pltpu.async_copy(hs_hbm.at[isv.at[j1]], r1, s1)
        pltpu.make_async_copy(hs_hbm.at[isv.at[j0]], r0, s0).wait()
        pltpu.sync_copy(r0, acc.at[idv.at[j0]], add=True)

        @pl.when(j0 + 2 < ROWS_PER_W)
        def _():
            pltpu.async_copy(hs_hbm.at[isv.at[j0 + 2]], r0, s0)

        cp1.wait()
        pltpu.sync_copy(r1, acc.at[idv.at[j1]], add=True)
        return carry

    lax.fori_loop(0, ROWS_PER_W // 2, body, 0)
    plsc.subcore_barrier()
    pltpu.sync_copy(acc.at[pl.ds(sid * RPT, RPT)],
                    out_hbm.at[cid, pl.ds(sid * RPT, RPT)])


def _sc_prop(hs, src2d, dst2d, D, tc_tiling):
    params = (None if tc_tiling
              else pltpu.CompilerParams(use_tc_tiling_on_sc=False))
    return pl.kernel(
        _prop_body,
        out_type=jax.ShapeDtypeStruct((2, NPAD, D), jnp.float32),
        mesh=_mesh,
        scratch_types=[
            pltpu.VMEM((ROWS_PER_W, 128), jnp.int32),
            pltpu.VMEM((ROWS_PER_W, 128), jnp.int32),
            pltpu.VMEM((128, D), jnp.float32),
            pltpu.VMEM((128, D), jnp.float32),
            pltpu.VMEM_SHARED((NPAD, D), jnp.float32),
            pltpu.SemaphoreType.DMA,
            pltpu.SemaphoreType.DMA,
        ],
        compiler_params=params,
    )(hs, src2d, dst2d)


def _mm_body(x_ref, w_ref, out_ref):
    out_ref[...] = jnp.dot(x_ref[...], w_ref[...],
                           preferred_element_type=jnp.float32)


def _tc_matmul(xp, W1):
    return pl.pallas_call(
        _mm_body,
        out_shape=jax.ShapeDtypeStruct((NPAD, 128), jnp.float32),
    )(xp, W1)


def _scale_body(h_ref, deg_ref, hs_ref, dinv_ref):
    deg = deg_ref[0, :, 0:1] + deg_ref[1, :, 0:1] + 1.0
    dinv = lax.rsqrt(deg)
    hs_ref[...] = h_ref[...] * dinv
    dinv_ref[...] = dinv


def _tc_scale(h1, degp):
    return pl.pallas_call(
        _scale_body,
        out_shape=[
            jax.ShapeDtypeStruct((NPAD, 128), jnp.float32),
            jax.ShapeDtypeStruct((NPAD, 1), jnp.float32),
        ],
    )(h1, degp)


def _tc_mid_body(a_ref, hs_ref, dinv_ref, b_ref, w_ref, out_ref):
    dinv = dinv_ref[...]
    p = jnp.tanh(dinv * (a_ref[0] + a_ref[1] - hs_ref[...]) + b_ref[...])
    out_ref[...] = jnp.dot(p, w_ref[...],
                           preferred_element_type=jnp.float32) * dinv


def _tc_mid(acc, hs, dinv, b, W, Dout):
    return pl.pallas_call(
        _tc_mid_body,
        out_shape=jax.ShapeDtypeStruct((NPAD, Dout), jnp.float32),
    )(acc, hs, dinv, b, W)


def _tc_final_body(a_ref, hs_ref, dinv_ref, b_ref, batch_ref, mask_ref,
                   wc_ref, bc_ref, out_ref, h_ref):
    dinv = dinv_ref[...]
    p = jnp.tanh(dinv * (a_ref[0] + a_ref[1] - hs_ref[...]) + b_ref[...])
    paug = jnp.concatenate([p, jnp.ones((NPAD, 1), jnp.float32)], axis=1)
    iota = lax.broadcasted_iota(jnp.int32, (NPAD, G), 1)
    onehot = (batch_ref[...] == iota).astype(jnp.float32)
    s = lax.dot_general(onehot, paug, (((0,), (0,)), ((), ())),
                        preferred_element_type=jnp.float32)
    hp = s[:, 0:16] / jnp.maximum(s[:, 16:17], 1.0)
    hd = mask_ref[...] * (2.0 * hp)
    h_ref[...] = hd
    out_ref[...] = jnp.dot(hd, wc_ref[...],
                           preferred_element_type=jnp.float32) + bc_ref[...]


def _tc_final(acc, hs, dinv, b, batch2d, mask, Wc, bc):
    return pl.pallas_call(
        _tc_final_body,
        out_shape=[
            jax.ShapeDtypeStruct((G, 2), jnp.float32),
            jax.ShapeDtypeStruct((G, 16), jnp.float32),
        ],
    )(acc, hs, dinv, b, batch2d, mask, Wc, bc)


def kernel(x, edge_index, batch, W1, b1, W2, b2, W3, b3, Wc, bc):
    xp = jnp.zeros((NPAD, 256), jnp.float32).at[:N].set(x)
    src = edge_index[0].astype(jnp.int32)
    dst = edge_index[1].astype(jnp.int32)
    npe = EPAD - E
    pad_idx = N + (jnp.arange(npe, dtype=jnp.int32) % (NPAD - N))
    src2d = jnp.concatenate([src, pad_idx]).reshape(EPAD // 128, 128)
    dst2d = jnp.concatenate([dst, pad_idx]).reshape(EPAD // 128, 128)
    batch2d = jnp.concatenate(
        [batch.astype(jnp.int32),
         jnp.full((NPAD - N,), G, jnp.int32)]).reshape(NPAD, 1)
    zeros16 = jnp.zeros((NPAD, 16), jnp.float32)
    ones16 = jnp.ones((128, 16), jnp.float32)
    mask = jax.random.bernoulli(jax.random.key(42), 0.5,
                                (G, 16)).astype(jnp.float32)
    b1r = b1.reshape(1, 128)
    b2r = b2.reshape(1, 64)
    b3r = b3.reshape(1, 16)
    bcr = bc.reshape(1, 2)

    degp = _sc_degree(dst2d, zeros16, ones16)
    h1 = _tc_matmul(xp, W1)
    hs1, dinv = _tc_scale(h1, degp)
    acc1 = _sc_prop(hs1, src2d, dst2d, 128, tc_tiling=True)
    hs2 = _tc_mid(acc1, hs1, dinv, b1r, W2, 64)
    acc2 = _sc_prop(hs2, src2d, dst2d, 64, tc_tiling=False)
    hs3 = _tc_mid(acc2, hs2, dinv, b2r, W3, 16)
    acc3 = _sc_prop(hs3, src2d, dst2d, 16, tc_tiling=False)
    out2d, h = _tc_final(acc3, hs3, dinv, b3r, batch2d, mask, Wc, bcr)
    return (out2d.reshape(-1), h)
